# baseline (speedup 1.0000x reference)
import functools

import jax
import jax.numpy as jnp
from jax import lax
from jax.experimental import pallas as pl
from jax.experimental.pallas import tpu as pltpu
from jax.experimental.pallas import tpu_sc as plsc

T = 2048
D = 768
E = 8
H = 1536
K = 2
P = T * K
BLK = 128
NBLK = P // BLK + E
R = NBLK * BLK
CH = 512
WLANES = 128
EOBPAD = 48

NC, NS = 2, 16
NW = NC * NS
PAIRS_W = P // NW
TOK_W = T // NW


def _router_body(x_ref, gw_ref, b_ref, wp_ref, p_ref, eob_ref, lb_ref):
    xx = x_ref[...]
    logits = jnp.dot(xx, gw_ref[...], preferred_element_type=jnp.float32)
    logits = logits + b_ref[...]
    sig = jax.nn.sigmoid(logits)
    scores = sig / (jnp.sum(sig, axis=-1, keepdims=True) + 1e-6)

    lane = lax.broadcasted_iota(jnp.int32, (T, E), 1)
    m1 = jnp.max(scores, axis=-1, keepdims=True)
    i1 = jnp.min(jnp.where(scores == m1, lane, E), axis=-1, keepdims=True)
    masked = jnp.where(lane == i1, -jnp.inf, scores)
    m2 = jnp.max(masked, axis=-1, keepdims=True)
    i2 = jnp.min(jnp.where(masked == m2, lane, E), axis=-1, keepdims=True)
    wsum = m1 + m2 + 1e-6
    w01 = jnp.concatenate([m1 / wsum, m2 / wsum], axis=0)
    wp_ref[...] = jnp.broadcast_to(w01, (P, 16))

    oh1 = (lane == i1).astype(jnp.float32)
    oh2 = (lane == i2).astype(jnp.float32)
    onehot = jnp.concatenate([oh1, oh2], axis=0)

    g = jnp.sum(onehot, axis=0, keepdims=True)
    avg_prob = jnp.mean(scores, axis=0, keepdims=True)
    lb_ref[...] = E * jnp.sum((g / T) * avg_prob, axis=1, keepdims=True)

    gi = g.astype(jnp.int32)
    blocks = (gi + (BLK - 1)) // BLK
    tri8 = (lax.broadcasted_iota(jnp.int32, (E, E), 0)
            <= lax.broadcasted_iota(jnp.int32, (E, E), 1)).astype(jnp.float32)
    cb = jnp.dot(blocks.astype(jnp.float32), tri8,
                 preferred_element_type=jnp.float32)
    off = (cb - blocks.astype(jnp.float32)) * BLK

    bi = lax.broadcasted_iota(jnp.int32, (EOBPAD, E), 0)
    eobv = jnp.sum((bi >= cb.astype(jnp.int32)).astype(jnp.int32),
                   axis=1, keepdims=True)
    eobv = jnp.minimum(eobv, E - 1)
    nreal = cb.astype(jnp.int32)[:, E - 1:E]
    rows1 = lax.broadcasted_iota(jnp.int32, (EOBPAD, 1), 0)
    eob_ref[...] = jnp.where(rows1 == NBLK, nreal, eobv)

    tri = (lax.broadcasted_iota(jnp.int32, (CH, CH), 0)
           > lax.broadcasted_iota(jnp.int32, (CH, CH), 1)).astype(jnp.float32)
    carry = jnp.zeros((1, E), jnp.float32)
    for c in range(P // CH):
        oc = onehot[c * CH:(c + 1) * CH]
        ranks = jnp.dot(tri, oc, preferred_element_type=jnp.float32) + carry
        pos = jnp.sum((ranks + off) * oc, axis=-1, keepdims=True)
        p_ref[c * CH:(c + 1) * CH, :] = pos.astype(jnp.int32)
        carry = carry + jnp.sum(oc, axis=0, keepdims=True)


_router = pl.pallas_call(
    _router_body,
    out_shape=(
        jax.ShapeDtypeStruct((P, 16), jnp.float32),
        jax.ShapeDtypeStruct((P, 1), jnp.int32),
        jax.ShapeDtypeStruct((EOBPAD, 1), jnp.int32),
        jax.ShapeDtypeStruct((1, 1), jnp.float32),
    ),
)


PW2 = PAIRS_W // 2


def _dispatch_body(x_hbm, p_hbm, xs_hbm,
                   idxa_v, idxb_v, rowsa_v, rowsb_v, sem, lsem):
    wid = lax.axis_index("s") * NC + lax.axis_index("c")
    jbase = wid * PAIRS_W
    tbase = lax.rem(jbase, T)
    lia = pltpu.async_copy(p_hbm.at[pl.ds(jbase, PW2)], idxa_v, lsem)
    lra = pltpu.async_copy(x_hbm.at[pl.ds(tbase, PW2)], rowsa_v, lsem)
    lia.wait()
    lra.wait()
    lib = pltpu.async_copy(p_hbm.at[pl.ds(jbase + PW2, PW2)], idxb_v, lsem)
    lrb = pltpu.async_copy(x_hbm.at[pl.ds(tbase + PW2, PW2)], rowsb_v, lsem)
    cpa = pltpu.async_copy(rowsa_v, xs_hbm.at[idxa_v], sem)
    lib.wait()
    lrb.wait()
    cpb = pltpu.async_copy(rowsb_v, xs_hbm.at[idxb_v], sem)
    cpa.wait()
    cpb.wait()


@functools.cache
def _dispatch():
    return pl.kernel(
        _dispatch_body,
        out_type=jax.ShapeDtypeStruct((R, D), jnp.float32),
        mesh=plsc.VectorSubcoreMesh(core_axis_name="c", subcore_axis_name="s",
                                    num_cores=NC, num_subcores=NS),
        scratch_types=[
            pltpu.VMEM((PW2,), jnp.int32),
            pltpu.VMEM((PW2,), jnp.int32),
            pltpu.VMEM((PW2, D), jnp.float32),
            pltpu.VMEM((PW2, D), jnp.float32),
            pltpu.SemaphoreType.DMA,
            pltpu.SemaphoreType.DMA,
        ],
    )


def _dot(a, b):
    return jnp.dot(a, b, preferred_element_type=jnp.float32,
                   precision=lax.Precision.DEFAULT)


def _gmm_body(eob_ref, xs_ref, wg_ref, wu_ref, wd_ref, y_ref):
    @pl.when(pl.program_id(0) < eob_ref[NBLK])
    def _():
        xb = xs_ref[...]
        hg = _dot(xb, wg_ref[0])
        hu = _dot(xb, wu_ref[0])
        act = hg * jax.nn.sigmoid(hg) * hu
        y_ref[...] = _dot(act, wd_ref[0])


_gmm = pl.pallas_call(
    _gmm_body,
    grid_spec=pltpu.PrefetchScalarGridSpec(
        num_scalar_prefetch=1,
        grid=(NBLK,),
        in_specs=[
            pl.BlockSpec((BLK, D), lambda b, eob: (jnp.where(b < eob[NBLK], b, 0), 0)),
            pl.BlockSpec((1, D, H), lambda b, eob: (eob[b], 0, 0)),
            pl.BlockSpec((1, D, H), lambda b, eob: (eob[b], 0, 0)),
            pl.BlockSpec((1, H, D), lambda b, eob: (eob[b], 0, 0)),
        ],
        out_specs=pl.BlockSpec((BLK, D),
                               lambda b, eob: (jnp.where(b < eob[NBLK], b, NBLK), 0)),
    ),
    out_shape=jax.ShapeDtypeStruct((R + BLK, D), jnp.float32),
)


TW2 = TOK_W // 2


def _combine_body(y_hbm, p_hbm, w_hbm, o_hbm, idx0a_v, idx1a_v, idx0b_v, idx1b_v,
                  w0a_v, w1a_v, w0b_v, w1b_v,
                  rows0a_v, rows1a_v, rows0b_v, rows1b_v, sem, wsem):
    wid = lax.axis_index("s") * NC + lax.axis_index("c")
    base = wid * TOK_W

    def add_rows(r0, r1, w0, w1):
        def add_token(t, carry):
            w0b = w0[t, :]
            w1b = w1[t, :]
            for v in range(D // 16):
                sl = pl.ds(v * 16, 16)
                r0[t, sl] = r0[t, sl] * w0b + r1[t, sl] * w1b
            return carry
        lax.fori_loop(0, TW2, add_token, 0)

    la0 = pltpu.async_copy(p_hbm.at[pl.ds(base, TW2)], idx0a_v, sem)
    la1 = pltpu.async_copy(p_hbm.at[pl.ds(T + base, TW2)], idx1a_v, sem)
    lb0 = pltpu.async_copy(p_hbm.at[pl.ds(base + TW2, TW2)], idx0b_v, sem)
    lb1 = pltpu.async_copy(p_hbm.at[pl.ds(T + base + TW2, TW2)], idx1b_v, sem)
    lw0a = pltpu.async_copy(w_hbm.at[pl.ds(base, TW2)], w0a_v, sem)
    lw1a = pltpu.async_copy(w_hbm.at[pl.ds(T + base, TW2)], w1a_v, sem)
    lw0b = pltpu.async_copy(w_hbm.at[pl.ds(base + TW2, TW2)], w0b_v, sem)
    lw1b = pltpu.async_copy(w_hbm.at[pl.ds(T + base + TW2, TW2)], w1b_v, sem)
    la0.wait()
    la1.wait()
    ga0 = pltpu.async_copy(y_hbm.at[idx0a_v], rows0a_v, sem)
    ga1 = pltpu.async_copy(y_hbm.at[idx1a_v], rows1a_v, sem)
    lb0.wait()
    lb1.wait()
    ga0.wait()
    ga1.wait()
    gb0 = pltpu.async_copy(y_hbm.at[idx0b_v], rows0b_v, sem)
    gb1 = pltpu.async_copy(y_hbm.at[idx1b_v], rows1b_v, sem)
    lw0a.wait()
    lw1a.wait()
    add_rows(rows0a_v, rows1a_v, w0a_v, w1a_v)
    wa = pltpu.async_copy(rows0a_v, o_hbm.at[pl.ds(base, TW2)], wsem)
    gb0.wait()
    gb1.wait()
    lw0b.wait()
    lw1b.wait()
    add_rows(rows0b_v, rows1b_v, w0b_v, w1b_v)
    wb = pltpu.async_copy(rows0b_v, o_hbm.at[pl.ds(base + TW2, TW2)], wsem)
    wa.wait()
    wb.wait()


@functools.cache
def _combine():
    return pl.kernel(
        _combine_body,
        out_type=jax.ShapeDtypeStruct((T, D), jnp.float32),
        mesh=plsc.VectorSubcoreMesh(core_axis_name="c", subcore_axis_name="s",
                                    num_cores=NC, num_subcores=NS),
        scratch_types=[
            pltpu.VMEM((TW2,), jnp.int32),
            pltpu.VMEM((TW2,), jnp.int32),
            pltpu.VMEM((TW2,), jnp.int32),
            pltpu.VMEM((TW2,), jnp.int32),
            pltpu.VMEM((TW2, 16), jnp.float32),
            pltpu.VMEM((TW2, 16), jnp.float32),
            pltpu.VMEM((TW2, 16), jnp.float32),
            pltpu.VMEM((TW2, 16), jnp.float32),
            pltpu.VMEM((TW2, D), jnp.float32),
            pltpu.VMEM((TW2, D), jnp.float32),
            pltpu.VMEM((TW2, D), jnp.float32),
            pltpu.VMEM((TW2, D), jnp.float32),
            pltpu.SemaphoreType.DMA,
            pltpu.SemaphoreType.DMA,
        ],
    )


def kernel(x, gate_W, expert_bias, W_gate, W_up, W_down):
    x2 = x.reshape(T, D)
    bias2 = expert_bias.reshape(1, E)
    wp, p, eob, lb = _router(x2, gate_W, bias2)
    p1 = p.reshape(P)
    xs = _dispatch()(x2, p1)
    y = _gmm(eob.reshape(EOBPAD), xs, W_gate, W_up, W_down)
    out = _combine()(y, p1, wp)
    return out.reshape(1, T, D), lb[0, 0]


# speedup vs baseline: 1.1577x; 1.1577x over previous
import functools

import jax
import jax.numpy as jnp
from jax import lax
from jax.experimental import pallas as pl
from jax.experimental.pallas import tpu as pltpu
from jax.experimental.pallas import tpu_sc as plsc

T = 2048
D = 768
E = 8
H = 1536
K = 2
P = T * K
BLK = 512
NBLK = P // BLK + E
R = NBLK * BLK
CH = 512
WLANES = 128
EOBPAD = 24

NC, NS = 2, 16
NW = NC * NS
PAIRS_W = P // NW
TOK_W = T // NW


def _router_body(x_ref, gw_ref, b_ref, wp_ref, p_ref, eob_ref, lb_ref):
    xx = x_ref[...]
    logits = jnp.dot(xx, gw_ref[...], preferred_element_type=jnp.float32)
    logits = logits + b_ref[...]
    sig = jax.nn.sigmoid(logits)
    scores = sig / (jnp.sum(sig, axis=-1, keepdims=True) + 1e-6)

    lane = lax.broadcasted_iota(jnp.int32, (T, E), 1)
    m1 = jnp.max(scores, axis=-1, keepdims=True)
    i1 = jnp.min(jnp.where(scores == m1, lane, E), axis=-1, keepdims=True)
    masked = jnp.where(lane == i1, -jnp.inf, scores)
    m2 = jnp.max(masked, axis=-1, keepdims=True)
    i2 = jnp.min(jnp.where(masked == m2, lane, E), axis=-1, keepdims=True)
    wsum = m1 + m2 + 1e-6
    w01 = jnp.concatenate([m1 / wsum, m2 / wsum], axis=0)
    wp_ref[...] = jnp.broadcast_to(w01, (P, 16))

    oh1 = (lane == i1).astype(jnp.float32)
    oh2 = (lane == i2).astype(jnp.float32)
    onehot = jnp.concatenate([oh1, oh2], axis=0)

    g = jnp.sum(onehot, axis=0, keepdims=True)
    avg_prob = jnp.mean(scores, axis=0, keepdims=True)
    lb_ref[...] = E * jnp.sum((g / T) * avg_prob, axis=1, keepdims=True)

    gi = g.astype(jnp.int32)
    blocks = (gi + (BLK - 1)) // BLK
    tri8 = (lax.broadcasted_iota(jnp.int32, (E, E), 0)
            <= lax.broadcasted_iota(jnp.int32, (E, E), 1)).astype(jnp.float32)
    cb = jnp.dot(blocks.astype(jnp.float32), tri8,
                 preferred_element_type=jnp.float32)
    off = (cb - blocks.astype(jnp.float32)) * BLK

    bi = lax.broadcasted_iota(jnp.int32, (EOBPAD, E), 0)
    eobv = jnp.sum((bi >= cb.astype(jnp.int32)).astype(jnp.int32),
                   axis=1, keepdims=True)
    eobv = jnp.minimum(eobv, E - 1)
    nreal = cb.astype(jnp.int32)[:, E - 1:E]
    rows1 = lax.broadcasted_iota(jnp.int32, (EOBPAD, 1), 0)
    eob_ref[...] = jnp.where(rows1 == NBLK, nreal, eobv)

    tri = (lax.broadcasted_iota(jnp.int32, (CH, CH), 0)
           > lax.broadcasted_iota(jnp.int32, (CH, CH), 1)).astype(jnp.float32)
    carry = jnp.zeros((1, E), jnp.float32)
    for c in range(P // CH):
        oc = onehot[c * CH:(c + 1) * CH]
        ranks = jnp.dot(tri, oc, preferred_element_type=jnp.float32) + carry
        pos = jnp.sum((ranks + off) * oc, axis=-1, keepdims=True)
        p_ref[c * CH:(c + 1) * CH, :] = pos.astype(jnp.int32)
        carry = carry + jnp.sum(oc, axis=0, keepdims=True)


_router = pl.pallas_call(
    _router_body,
    out_shape=(
        jax.ShapeDtypeStruct((P, 16), jnp.float32),
        jax.ShapeDtypeStruct((P, 1), jnp.int32),
        jax.ShapeDtypeStruct((EOBPAD, 1), jnp.int32),
        jax.ShapeDtypeStruct((1, 1), jnp.float32),
    ),
)


PW2 = PAIRS_W // 2


def _dispatch_body(x_hbm, p_hbm, xs_hbm,
                   idxa_v, idxb_v, rowsa_v, rowsb_v, sem, lsem):
    wid = lax.axis_index("s") * NC + lax.axis_index("c")
    jbase = wid * PAIRS_W
    tbase = lax.rem(jbase, T)
    lia = pltpu.async_copy(p_hbm.at[pl.ds(jbase, PW2)], idxa_v, lsem)
    lra = pltpu.async_copy(x_hbm.at[pl.ds(tbase, PW2)], rowsa_v, lsem)
    lia.wait()
    lra.wait()
    lib = pltpu.async_copy(p_hbm.at[pl.ds(jbase + PW2, PW2)], idxb_v, lsem)
    lrb = pltpu.async_copy(x_hbm.at[pl.ds(tbase + PW2, PW2)], rowsb_v, lsem)
    cpa = pltpu.async_copy(rowsa_v, xs_hbm.at[idxa_v], sem)
    lib.wait()
    lrb.wait()
    cpb = pltpu.async_copy(rowsb_v, xs_hbm.at[idxb_v], sem)
    cpa.wait()
    cpb.wait()


@functools.cache
def _dispatch():
    return pl.kernel(
        _dispatch_body,
        out_type=jax.ShapeDtypeStruct((R, D), jnp.float32),
        mesh=plsc.VectorSubcoreMesh(core_axis_name="c", subcore_axis_name="s",
                                    num_cores=NC, num_subcores=NS),
        scratch_types=[
            pltpu.VMEM((PW2,), jnp.int32),
            pltpu.VMEM((PW2,), jnp.int32),
            pltpu.VMEM((PW2, D), jnp.float32),
            pltpu.VMEM((PW2, D), jnp.float32),
            pltpu.SemaphoreType.DMA,
            pltpu.SemaphoreType.DMA,
        ],
    )


def _dot(a, b):
    return jnp.dot(a, b, preferred_element_type=jnp.float32,
                   precision=lax.Precision.DEFAULT)


def _gmm_body(eob_ref, xs_ref, wg_ref, wu_ref, wd_ref, y_ref):
    @pl.when(pl.program_id(0) < eob_ref[NBLK])
    def _():
        xb = xs_ref[...]
        hg = _dot(xb, wg_ref[0])
        hu = _dot(xb, wu_ref[0])
        act = hg * jax.nn.sigmoid(hg) * hu
        y_ref[...] = _dot(act, wd_ref[0])


_gmm = pl.pallas_call(
    _gmm_body,
    grid_spec=pltpu.PrefetchScalarGridSpec(
        num_scalar_prefetch=1,
        grid=(NBLK,),
        in_specs=[
            pl.BlockSpec((BLK, D), lambda b, eob: (jnp.where(b < eob[NBLK], b, 0), 0)),
            pl.BlockSpec((1, D, H), lambda b, eob: (eob[b], 0, 0)),
            pl.BlockSpec((1, D, H), lambda b, eob: (eob[b], 0, 0)),
            pl.BlockSpec((1, H, D), lambda b, eob: (eob[b], 0, 0)),
        ],
        out_specs=pl.BlockSpec((BLK, D),
                               lambda b, eob: (jnp.where(b < eob[NBLK], b, NBLK), 0)),
    ),
    out_shape=jax.ShapeDtypeStruct((R + BLK, D), jnp.float32),
)


TW2 = TOK_W // 2


def _combine_body(y_hbm, p_hbm, w_hbm, o_hbm, idx0a_v, idx1a_v, idx0b_v, idx1b_v,
                  w0a_v, w1a_v, w0b_v, w1b_v,
                  rows0a_v, rows1a_v, rows0b_v, rows1b_v, sem, wsem):
    wid = lax.axis_index("s") * NC + lax.axis_index("c")
    base = wid * TOK_W

    def add_rows(r0, r1, w0, w1):
        def add_token(t, carry):
            w0b = w0[t, :]
            w1b = w1[t, :]
            for v in range(D // 16):
                sl = pl.ds(v * 16, 16)
                r0[t, sl] = r0[t, sl] * w0b + r1[t, sl] * w1b
            return carry
        lax.fori_loop(0, TW2, add_token, 0)

    la0 = pltpu.async_copy(p_hbm.at[pl.ds(base, TW2)], idx0a_v, sem)
    la1 = pltpu.async_copy(p_hbm.at[pl.ds(T + base, TW2)], idx1a_v, sem)
    lb0 = pltpu.async_copy(p_hbm.at[pl.ds(base + TW2, TW2)], idx0b_v, sem)
    lb1 = pltpu.async_copy(p_hbm.at[pl.ds(T + base + TW2, TW2)], idx1b_v, sem)
    lw0a = pltpu.async_copy(w_hbm.at[pl.ds(base, TW2)], w0a_v, sem)
    lw1a = pltpu.async_copy(w_hbm.at[pl.ds(T + base, TW2)], w1a_v, sem)
    lw0b = pltpu.async_copy(w_hbm.at[pl.ds(base + TW2, TW2)], w0b_v, sem)
    lw1b = pltpu.async_copy(w_hbm.at[pl.ds(T + base + TW2, TW2)], w1b_v, sem)
    la0.wait()
    la1.wait()
    ga0 = pltpu.async_copy(y_hbm.at[idx0a_v], rows0a_v, sem)
    ga1 = pltpu.async_copy(y_hbm.at[idx1a_v], rows1a_v, sem)
    lb0.wait()
    lb1.wait()
    ga0.wait()
    ga1.wait()
    gb0 = pltpu.async_copy(y_hbm.at[idx0b_v], rows0b_v, sem)
    gb1 = pltpu.async_copy(y_hbm.at[idx1b_v], rows1b_v, sem)
    lw0a.wait()
    lw1a.wait()
    add_rows(rows0a_v, rows1a_v, w0a_v, w1a_v)
    wa = pltpu.async_copy(rows0a_v, o_hbm.at[pl.ds(base, TW2)], wsem)
    gb0.wait()
    gb1.wait()
    lw0b.wait()
    lw1b.wait()
    add_rows(rows0b_v, rows1b_v, w0b_v, w1b_v)
    wb = pltpu.async_copy(rows0b_v, o_hbm.at[pl.ds(base + TW2, TW2)], wsem)
    wa.wait()
    wb.wait()


@functools.cache
def _combine():
    return pl.kernel(
        _combine_body,
        out_type=jax.ShapeDtypeStruct((T, D), jnp.float32),
        mesh=plsc.VectorSubcoreMesh(core_axis_name="c", subcore_axis_name="s",
                                    num_cores=NC, num_subcores=NS),
        scratch_types=[
            pltpu.VMEM((TW2,), jnp.int32),
            pltpu.VMEM((TW2,), jnp.int32),
            pltpu.VMEM((TW2,), jnp.int32),
            pltpu.VMEM((TW2,), jnp.int32),
            pltpu.VMEM((TW2, 16), jnp.float32),
            pltpu.VMEM((TW2, 16), jnp.float32),
            pltpu.VMEM((TW2, 16), jnp.float32),
            pltpu.VMEM((TW2, 16), jnp.float32),
            pltpu.VMEM((TW2, D), jnp.float32),
            pltpu.VMEM((TW2, D), jnp.float32),
            pltpu.VMEM((TW2, D), jnp.float32),
            pltpu.VMEM((TW2, D), jnp.float32),
            pltpu.SemaphoreType.DMA,
            pltpu.SemaphoreType.DMA,
        ],
    )


def kernel(x, gate_W, expert_bias, W_gate, W_up, W_down):
    x2 = x.reshape(T, D)
    bias2 = expert_bias.reshape(1, E)
    wp, p, eob, lb = _router(x2, gate_W, bias2)
    p1 = p.reshape(P)
    xs = _dispatch()(x2, p1)
    y = _gmm(eob.reshape(EOBPAD), xs, W_gate, W_up, W_down)
    out = _combine()(y, p1, wp)
    return out.reshape(1, T, D), lb[0, 0]


# speedup vs baseline: 1.1593x; 1.0014x over previous
import functools

import jax
import jax.numpy as jnp
from jax import lax
from jax.experimental import pallas as pl
from jax.experimental.pallas import tpu as pltpu
from jax.experimental.pallas import tpu_sc as plsc

T = 2048
D = 768
E = 8
H = 1536
K = 2
P = T * K
BLK = 512
NBLK = P // BLK + E
R = NBLK * BLK
CH = 512
EOBPAD = 24

NC, NS = 2, 16
NW = NC * NS
PAIRS_W = P // NW
TOK_W = T // NW


def _router_body(x_ref, gw_ref, b_ref, wp_ref, p_ref, eob_ref, lb_ref):
    xx = x_ref[...]
    logits = jnp.dot(xx, gw_ref[...], preferred_element_type=jnp.float32)
    logits = logits + b_ref[...]
    sig = jax.nn.sigmoid(logits)
    scores = sig / (jnp.sum(sig, axis=-1, keepdims=True) + 1e-6)

    lane = lax.broadcasted_iota(jnp.int32, (T, E), 1)
    m1 = jnp.max(scores, axis=-1, keepdims=True)
    i1 = jnp.min(jnp.where(scores == m1, lane, E), axis=-1, keepdims=True)
    masked = jnp.where(lane == i1, -jnp.inf, scores)
    m2 = jnp.max(masked, axis=-1, keepdims=True)
    i2 = jnp.min(jnp.where(masked == m2, lane, E), axis=-1, keepdims=True)
    wsum = m1 + m2 + 1e-6
    w01 = jnp.concatenate([m1 / wsum, m2 / wsum], axis=0)
    wp_ref[...] = jnp.broadcast_to(w01, (P, 16))

    oh1 = (lane == i1).astype(jnp.float32)
    oh2 = (lane == i2).astype(jnp.float32)
    onehot = jnp.concatenate([oh1, oh2], axis=0)

    g = jnp.sum(onehot, axis=0, keepdims=True)
    avg_prob = jnp.mean(scores, axis=0, keepdims=True)
    lb_ref[...] = E * jnp.sum((g / T) * avg_prob, axis=1, keepdims=True)

    gi = g.astype(jnp.int32)
    blocks = (gi + (BLK - 1)) // BLK
    tri8 = (lax.broadcasted_iota(jnp.int32, (E, E), 0)
            <= lax.broadcasted_iota(jnp.int32, (E, E), 1)).astype(jnp.float32)
    cb = jnp.dot(blocks.astype(jnp.float32), tri8,
                 preferred_element_type=jnp.float32)
    off = (cb - blocks.astype(jnp.float32)) * BLK

    bi = lax.broadcasted_iota(jnp.int32, (EOBPAD, E), 0)
    eobv = jnp.sum((bi >= cb.astype(jnp.int32)).astype(jnp.int32),
                   axis=1, keepdims=True)
    eobv = jnp.minimum(eobv, E - 1)
    nreal = cb.astype(jnp.int32)[:, E - 1:E]
    rows1 = lax.broadcasted_iota(jnp.int32, (EOBPAD, 1), 0)
    eob_ref[...] = jnp.where(rows1 == NBLK, nreal, eobv)

    tri = (lax.broadcasted_iota(jnp.int32, (CH, CH), 0)
           > lax.broadcasted_iota(jnp.int32, (CH, CH), 1)).astype(jnp.float32)
    carry = jnp.zeros((1, E), jnp.float32)
    for c in range(P // CH):
        oc = onehot[c * CH:(c + 1) * CH]
        ranks = jnp.dot(tri, oc, preferred_element_type=jnp.float32) + carry
        pos = jnp.sum((ranks + off) * oc, axis=-1, keepdims=True)
        p_ref[c * CH:(c + 1) * CH, :] = pos.astype(jnp.int32)
        carry = carry + jnp.sum(oc, axis=0, keepdims=True)


_router = pl.pallas_call(
    _router_body,
    out_shape=(
        jax.ShapeDtypeStruct((P, 16), jnp.float32),
        jax.ShapeDtypeStruct((P, 1), jnp.int32),
        jax.ShapeDtypeStruct((EOBPAD, 1), jnp.int32),
        jax.ShapeDtypeStruct((1, 1), jnp.float32),
    ),
)


PW2 = PAIRS_W // 2


def _dispatch_body(x_hbm, p_hbm, xs_hbm,
                   idxa_v, idxb_v, rowsa_v, rowsb_v, sem, lsem):
    wid = lax.axis_index("s") * NC + lax.axis_index("c")
    jbase = wid * PAIRS_W
    tbase = lax.rem(jbase, T)
    lia = pltpu.async_copy(p_hbm.at[pl.ds(jbase, PW2)], idxa_v, lsem)
    lra = pltpu.async_copy(x_hbm.at[pl.ds(tbase, PW2)], rowsa_v, lsem)
    lia.wait()
    lra.wait()
    lib = pltpu.async_copy(p_hbm.at[pl.ds(jbase + PW2, PW2)], idxb_v, lsem)
    lrb = pltpu.async_copy(x_hbm.at[pl.ds(tbase + PW2, PW2)], rowsb_v, lsem)
    cpa = pltpu.async_copy(rowsa_v, xs_hbm.at[idxa_v], sem)
    lib.wait()
    lrb.wait()
    cpb = pltpu.async_copy(rowsb_v, xs_hbm.at[idxb_v], sem)
    cpa.wait()
    cpb.wait()


@functools.cache
def _dispatch():
    return pl.kernel(
        _dispatch_body,
        out_type=jax.ShapeDtypeStruct((R, D), jnp.float32),
        mesh=plsc.VectorSubcoreMesh(core_axis_name="c", subcore_axis_name="s",
                                    num_cores=NC, num_subcores=NS),
        scratch_types=[
            pltpu.VMEM((PW2,), jnp.int32),
            pltpu.VMEM((PW2,), jnp.int32),
            pltpu.VMEM((PW2, D), jnp.float32),
            pltpu.VMEM((PW2, D), jnp.float32),
            pltpu.SemaphoreType.DMA,
            pltpu.SemaphoreType.DMA,
        ],
    )


def _dot(a, b):
    return jnp.dot(a, b, preferred_element_type=jnp.float32,
                   precision=lax.Precision.DEFAULT)


def _gmm_body(eob_ref, xs_ref, wg_ref, wu_ref, wd_ref, y_ref):
    @pl.when(pl.program_id(0) < eob_ref[NBLK])
    def _():
        xb = xs_ref[...]
        hg = _dot(xb, wg_ref[0])
        hu = _dot(xb, wu_ref[0])
        act = hg * jax.nn.sigmoid(hg) * hu
        y_ref[...] = _dot(act, wd_ref[0])


_gmm = pl.pallas_call(
    _gmm_body,
    grid_spec=pltpu.PrefetchScalarGridSpec(
        num_scalar_prefetch=1,
        grid=(NBLK,),
        in_specs=[
            pl.BlockSpec((BLK, D), lambda b, eob: (jnp.where(b < eob[NBLK], b, 0), 0)),
            pl.BlockSpec((1, D, H), lambda b, eob: (eob[b], 0, 0)),
            pl.BlockSpec((1, D, H), lambda b, eob: (eob[b], 0, 0)),
            pl.BlockSpec((1, H, D), lambda b, eob: (eob[b], 0, 0)),
        ],
        out_specs=pl.BlockSpec((BLK, D),
                               lambda b, eob: (jnp.where(b < eob[NBLK], b, NBLK), 0)),
    ),
    out_shape=jax.ShapeDtypeStruct((R + BLK, D), jnp.float32),
)


TW2 = TOK_W // 2


def _combine_body(y_hbm, p_hbm, w_hbm, o_hbm, idx0a_v, idx1a_v, idx0b_v, idx1b_v,
                  w0a_v, w1a_v, w0b_v, w1b_v,
                  rows0a_v, rows1a_v, rows0b_v, rows1b_v, sem, wsem):
    wid = lax.axis_index("s") * NC + lax.axis_index("c")
    base = wid * TOK_W

    def add_rows(r0, r1, w0, w1):
        def add_token(t, carry):
            w0b = w0[t, :]
            w1b = w1[t, :]
            for v in range(D // 16):
                sl = pl.ds(v * 16, 16)
                r0[t, sl] = r0[t, sl] * w0b + r1[t, sl] * w1b
            return carry
        lax.fori_loop(0, TW2, add_token, 0)

    la0 = pltpu.async_copy(p_hbm.at[pl.ds(base, TW2)], idx0a_v, sem)
    la1 = pltpu.async_copy(p_hbm.at[pl.ds(T + base, TW2)], idx1a_v, sem)
    lb0 = pltpu.async_copy(p_hbm.at[pl.ds(base + TW2, TW2)], idx0b_v, sem)
    lb1 = pltpu.async_copy(p_hbm.at[pl.ds(T + base + TW2, TW2)], idx1b_v, sem)
    lw0a = pltpu.async_copy(w_hbm.at[pl.ds(base, TW2)], w0a_v, sem)
    lw1a = pltpu.async_copy(w_hbm.at[pl.ds(T + base, TW2)], w1a_v, sem)
    lw0b = pltpu.async_copy(w_hbm.at[pl.ds(base + TW2, TW2)], w0b_v, sem)
    lw1b = pltpu.async_copy(w_hbm.at[pl.ds(T + base + TW2, TW2)], w1b_v, sem)
    la0.wait()
    la1.wait()
    ga0 = pltpu.async_copy(y_hbm.at[idx0a_v], rows0a_v, sem)
    ga1 = pltpu.async_copy(y_hbm.at[idx1a_v], rows1a_v, sem)
    lb0.wait()
    lb1.wait()
    ga0.wait()
    ga1.wait()
    gb0 = pltpu.async_copy(y_hbm.at[idx0b_v], rows0b_v, sem)
    gb1 = pltpu.async_copy(y_hbm.at[idx1b_v], rows1b_v, sem)
    lw0a.wait()
    lw1a.wait()
    add_rows(rows0a_v, rows1a_v, w0a_v, w1a_v)
    wa = pltpu.async_copy(rows0a_v, o_hbm.at[pl.ds(base, TW2)], wsem)
    gb0.wait()
    gb1.wait()
    lw0b.wait()
    lw1b.wait()
    add_rows(rows0b_v, rows1b_v, w0b_v, w1b_v)
    wb = pltpu.async_copy(rows0b_v, o_hbm.at[pl.ds(base + TW2, TW2)], wsem)
    wa.wait()
    wb.wait()


@functools.cache
def _combine():
    return pl.kernel(
        _combine_body,
        out_type=jax.ShapeDtypeStruct((T, D), jnp.float32),
        mesh=plsc.VectorSubcoreMesh(core_axis_name="c", subcore_axis_name="s",
                                    num_cores=NC, num_subcores=NS),
        scratch_types=[
            pltpu.VMEM((TW2,), jnp.int32),
            pltpu.VMEM((TW2,), jnp.int32),
            pltpu.VMEM((TW2,), jnp.int32),
            pltpu.VMEM((TW2,), jnp.int32),
            pltpu.VMEM((TW2, 16), jnp.float32),
            pltpu.VMEM((TW2, 16), jnp.float32),
            pltpu.VMEM((TW2, 16), jnp.float32),
            pltpu.VMEM((TW2, 16), jnp.float32),
            pltpu.VMEM((TW2, D), jnp.float32),
            pltpu.VMEM((TW2, D), jnp.float32),
            pltpu.VMEM((TW2, D), jnp.float32),
            pltpu.VMEM((TW2, D), jnp.float32),
            pltpu.SemaphoreType.DMA,
            pltpu.SemaphoreType.DMA,
        ],
    )


def kernel(x, gate_W, expert_bias, W_gate, W_up, W_down):
    x2 = x.reshape(T, D)
    bias2 = expert_bias.reshape(1, E)
    wp, p, eob, lb = _router(x2, gate_W, bias2)
    p1 = p.reshape(P)
    xs = _dispatch()(x2, p1)
    y = _gmm(eob.reshape(EOBPAD), xs, W_gate, W_up, W_down)
    out = _combine()(y, p1, wp)
    return out.reshape(1, T, D), lb[0, 0]
